# baseline (speedup 1.0000x reference)
import functools

import jax
import jax.numpy as jnp
from jax import lax
from jax.experimental import pallas as pl
from jax.experimental.pallas import tpu as pltpu
from jax.experimental.pallas import tpu_sc as plsc

N = 10000
E = 320000
D = 128

NC = 2
NS = 16
NW = NC * NS
EPW = E // NW
CH = 48
NCHUNK = 212
EPT = NCHUNK * CH
EPAD = NW * EPT - E
NBUF = 4
PD = 2
SD = 2
NP = 10240
RPT = NP // NS

_PREC = jax.lax.Precision.DEFAULT


def _deg_body(dst_hbm, out_hbm, dst_v, deg_v, acc_v, tmp_v, deg_sh):
    cid = lax.axis_index("c")
    sid = lax.axis_index("s")
    wid = sid * NC + cid
    zeros16 = jnp.zeros((16,), jnp.float32)
    ones16 = jnp.ones((16,), jnp.float32)

    def zbody(i, c):
        deg_v[pl.ds(pl.multiple_of(i * 16, 16), 16)] = zeros16
        return c
    lax.fori_loop(0, NP // 16, zbody, 0)

    pltpu.sync_copy(dst_hbm.at[pl.ds(wid * EPW, EPW)], dst_v)

    def ebody(i, c):
        iv = dst_v[pl.ds(pl.multiple_of(i * 16, 16), 16)]
        plsc.addupdate_scatter(deg_v, [iv], ones16)
        return c
    lax.fori_loop(0, EPW // 16, ebody, 0)

    pltpu.sync_copy(deg_v, deg_sh.at[sid])
    plsc.subcore_barrier()

    pltpu.sync_copy(deg_sh.at[0, pl.ds(sid * RPT, RPT)], acc_v)

    def mbody(j, c):
        pltpu.sync_copy(deg_sh.at[j, pl.ds(sid * RPT, RPT)], tmp_v)

        def abody(i, c2):
            sl = pl.ds(pl.multiple_of(i * 16, 16), 16)
            acc_v[sl] = acc_v[sl] + tmp_v[sl]
            return c2
        lax.fori_loop(0, RPT // 16, abody, 0)
        return c
    lax.fori_loop(1, NS, mbody, 0)
    pltpu.sync_copy(acc_v, out_hbm.at[pl.ds(cid * NP + sid * RPT, RPT)])


def _agg_body(xws_hbm, srcf_hbm, dstf_hbm, zeros_hbm, out_hbm,
              srci_v, dsti_v, rows0, rows1, rows2, rows3,
              db0, db1, db2, db3, acc_sh,
              g0, g1, g2, g3, s0, s1, s2, s3):
    rows = [rows0, rows1, rows2, rows3]
    dbuf = [db0, db1, db2, db3]
    gsem = [g0, g1, g2, g3]
    ssem = [s0, s1, s2, s3]
    cid = lax.axis_index("c")
    sid = lax.axis_index("s")
    wid = sid * NC + cid
    pltpu.sync_copy(zeros_hbm.at[pl.ds(sid * RPT, RPT)],
                    acc_sh.at[pl.ds(sid * RPT, RPT)])
    pltpu.sync_copy(srcf_hbm.at[pl.ds(wid * EPT, EPT)], srci_v)
    pltpu.sync_copy(dstf_hbm.at[pl.ds(wid * EPT, EPT)], dsti_v)
    plsc.subcore_barrier()

    for b in range(PD):
        pltpu.async_copy(xws_hbm.at[srci_v.at[pl.ds(b * CH, CH)]],
                         rows[b], gsem[b])

    def group(g, carry):
        for b in range(NBUF):
            c = g * NBUF + b
            pltpu.make_async_copy(xws_hbm.at[srci_v.at[pl.ds(0, CH)]],
                                  rows[b], gsem[b]).wait()
            for k in range(CH // 16):
                off = pl.multiple_of(c * CH + k * 16, 8)
                dbuf[b][pl.ds(k * 16, 16)] = dsti_v[pl.ds(off, 16)]
            pltpu.async_copy(rows[b], acc_sh.at[dbuf[b]], ssem[b], add=True)

            @pl.when(c >= SD)
            def _wait_prev_scatter():
                sb = (b + NBUF - SD) % NBUF
                pltpu.make_async_copy(rows[sb], acc_sh.at[dbuf[sb]],
                                      ssem[sb]).wait()

            @pl.when(c + PD < NCHUNK)
            def _issue_next_gather():
                nb = (b + PD) % NBUF
                off = pl.multiple_of((c + PD) * CH, 8)
                pltpu.async_copy(xws_hbm.at[srci_v.at[pl.ds(off, CH)]],
                                 rows[nb], gsem[nb])
        return carry

    lax.fori_loop(0, NCHUNK // NBUF, group, 0)
    for k in range(SD):
        b = (NCHUNK - SD + k) % NBUF
        pltpu.make_async_copy(rows[b], acc_sh.at[dbuf[b]], ssem[b]).wait()
    plsc.subcore_barrier()
    pltpu.sync_copy(acc_sh.at[pl.ds(sid * RPT, RPT)],
                    out_hbm.at[pl.ds(cid * NP + sid * RPT, RPT)])


def _xw_body(nf_ref, wg_ref, wih_ref, whh_ref, bih_ref, bhh_ref, xw_ref):
    W = wg_ref[...]
    gi = lax.dot_general(W, wih_ref[...], (((1,), (1,)), ((), ())),
                         precision=_PREC) + bih_ref[...]
    gh = lax.dot_general(W, whh_ref[...], (((1,), (1,)), ((), ())),
                         precision=_PREC) + bhh_ref[...]
    r = jax.nn.sigmoid(gi[:, :D] + gh[:, :D])
    z = jax.nn.sigmoid(gi[:, D:2 * D] + gh[:, D:2 * D])
    n = jnp.tanh(gi[:, 2 * D:] + r * gh[:, 2 * D:])
    w_ev = (1.0 - z) * n + z * W
    xw_ref[...] = jnp.dot(nf_ref[...], w_ev, precision=_PREC)


_xw_call = pl.pallas_call(
    _xw_body,
    out_shape=jax.ShapeDtypeStruct((N, D), jnp.float32),
)


def _scale_body(xw_ref, degp_ref, xws_ref):
    deg = degp_ref[0, :N] + degp_ref[1, :N] + 1.0
    xws_ref[:N] = xw_ref[...] * lax.rsqrt(deg)
    xws_ref[N:] = jnp.zeros((NP - N, D), jnp.float32)


_scale_call = pl.pallas_call(
    _scale_body,
    out_shape=jax.ShapeDtypeStruct((NP, D), jnp.float32),
)


def _out_body(acc_ref, xws_ref, degp_ref, wlin_ref, blin_ref, out_ref):
    deg = degp_ref[0, :N] + degp_ref[1, :N] + 1.0
    h = (acc_ref[0, :N] + acc_ref[1, :N] + xws_ref[:N]) * lax.rsqrt(deg)
    zr = jnp.maximum(h, 0.0)
    out_ref[...] = lax.dot_general(zr, wlin_ref[...], (((1,), (1,)), ((), ())),
                                   precision=_PREC) + blin_ref[...]


_out_call = pl.pallas_call(
    _out_body,
    out_shape=jax.ShapeDtypeStruct((N, D), jnp.float32),
)


@functools.lru_cache(maxsize=1)
def _sc_kernels():
    mesh = plsc.VectorSubcoreMesh(core_axis_name="c", subcore_axis_name="s")
    deg_kernel = pl.kernel(
        _deg_body,
        out_type=jax.ShapeDtypeStruct((NC * NP,), jnp.float32),
        mesh=mesh,
        compiler_params=pltpu.CompilerParams(needs_layout_passes=False),
        scratch_types=[
            pltpu.VMEM((EPW,), jnp.int32),
            pltpu.VMEM((NP,), jnp.float32),
            pltpu.VMEM((RPT,), jnp.float32),
            pltpu.VMEM((RPT,), jnp.float32),
            pltpu.VMEM_SHARED((NS, NP), jnp.float32),
        ],
    )
    agg_kernel = pl.kernel(
        _agg_body,
        out_type=jax.ShapeDtypeStruct((NC * NP, D), jnp.float32),
        mesh=mesh,
        compiler_params=pltpu.CompilerParams(needs_layout_passes=False),
        scratch_types=(
            [pltpu.VMEM((EPT,), jnp.int32),
             pltpu.VMEM((EPT,), jnp.int32)]
            + [pltpu.VMEM((CH, D), jnp.float32) for _ in range(NBUF)]
            + [pltpu.VMEM((CH,), jnp.int32) for _ in range(NBUF)]
            + [pltpu.VMEM_SHARED((NP, D), jnp.float32)]
            + [pltpu.SemaphoreType.DMA] * (2 * NBUF)
        ),
    )
    return deg_kernel, agg_kernel


def kernel(node_feat, src, dst, W_gcn, W_ih, W_hh, b_ih, b_hh, W_lin, b_lin):
    src = src.astype(jnp.int32)
    dst = dst.astype(jnp.int32)
    zeros_acc = jnp.zeros((NP, D), jnp.float32)

    srcf = jnp.pad(src, (0, EPAD), constant_values=NP - 1)
    dstf = jnp.pad(dst, (0, EPAD), constant_values=NP - 1)

    _deg_kernel, _agg_kernel = _sc_kernels()
    degp = _deg_kernel(dst).reshape(NC, NP, 1)
    xw = _xw_call(node_feat.astype(jnp.float32), W_gcn, W_ih, W_hh,
                  b_ih.reshape(1, 3 * D), b_hh.reshape(1, 3 * D))
    xws = _scale_call(xw, degp)
    acc = _agg_kernel(xws, srcf, dstf, zeros_acc).reshape(NC, NP, D)
    return _out_call(acc, xws, degp, W_lin, b_lin.reshape(1, D))


# speedup vs baseline: 1.2732x; 1.2732x over previous
import functools

import jax
import jax.numpy as jnp
from jax import lax
from jax.experimental import pallas as pl
from jax.experimental.pallas import tpu as pltpu
from jax.experimental.pallas import tpu_sc as plsc

N = 10000
E = 320000
D = 128

NC = 2
NS = 16
NW = NC * NS
EPW = E // NW
CH = 48
NCHUNK = 212
EPT = NCHUNK * CH
EPAD = NW * EPT - E
NBUF = 4
PD = 2
SD = 2
NP = 10240
RPT = NP // NS

_PREC = jax.lax.Precision.DEFAULT


def _deg_body(dst_hbm, out_hbm, dst_v, deg_v, acc_v, tmp_v, deg_sh):
    cid = lax.axis_index("c")
    sid = lax.axis_index("s")
    wid = sid * NC + cid
    zeros16 = jnp.zeros((16,), jnp.float32)
    ones16 = jnp.ones((16,), jnp.float32)

    def zbody(i, c):
        deg_v[pl.ds(pl.multiple_of(i * 16, 16), 16)] = zeros16
        return c
    lax.fori_loop(0, NP // 16, zbody, 0)

    pltpu.sync_copy(dst_hbm.at[pl.ds(wid * EPW, EPW)], dst_v)

    def ebody(i, c):
        iv = dst_v[pl.ds(pl.multiple_of(i * 16, 16), 16)]
        plsc.addupdate_scatter(deg_v, [iv], ones16)
        return c
    lax.fori_loop(0, EPW // 16, ebody, 0)

    pltpu.sync_copy(deg_v, deg_sh.at[sid])
    plsc.subcore_barrier()

    pltpu.sync_copy(deg_sh.at[0, pl.ds(sid * RPT, RPT)], acc_v)

    def mbody(j, c):
        pltpu.sync_copy(deg_sh.at[j, pl.ds(sid * RPT, RPT)], tmp_v)

        def abody(i, c2):
            sl = pl.ds(pl.multiple_of(i * 16, 16), 16)
            acc_v[sl] = acc_v[sl] + tmp_v[sl]
            return c2
        lax.fori_loop(0, RPT // 16, abody, 0)
        return c
    lax.fori_loop(1, NS, mbody, 0)
    pltpu.sync_copy(acc_v, out_hbm.at[pl.ds(cid * NP + sid * RPT, RPT)])


def _agg_body(xws_hbm, srcf_hbm, dstf_hbm, zeros_hbm, out_hbm,
              srci_v, dsti_v, rows0, rows1, rows2, rows3,
              db0, db1, db2, db3, acc_sh,
              g0, g1, g2, g3, s0, s1, s2, s3):
    rows = [rows0, rows1, rows2, rows3]
    dbuf = [db0, db1, db2, db3]
    gsem = [g0, g1, g2, g3]
    ssem = [s0, s1, s2, s3]
    cid = lax.axis_index("c")
    sid = lax.axis_index("s")
    wid = sid * NC + cid
    pltpu.sync_copy(zeros_hbm.at[pl.ds(sid * RPT, RPT)],
                    acc_sh.at[pl.ds(sid * RPT, RPT)])
    pltpu.sync_copy(srcf_hbm.at[pl.ds(wid * EPT, EPT)], srci_v)
    pltpu.sync_copy(dstf_hbm.at[pl.ds(wid * EPT, EPT)], dsti_v)
    plsc.subcore_barrier()

    for b in range(PD):
        pltpu.async_copy(xws_hbm.at[srci_v.at[pl.ds(b * CH, CH)]],
                         rows[b], gsem[b])

    def group(g, carry):
        for b in range(NBUF):
            c = g * NBUF + b
            pltpu.make_async_copy(xws_hbm.at[srci_v.at[pl.ds(0, CH)]],
                                  rows[b], gsem[b]).wait()
            for k in range(CH // 16):
                off = pl.multiple_of(c * CH + k * 16, 8)
                dbuf[b][pl.ds(k * 16, 16)] = dsti_v[pl.ds(off, 16)]
            pltpu.async_copy(rows[b], acc_sh.at[dbuf[b]], ssem[b], add=True)

            @pl.when(c >= SD)
            def _wait_prev_scatter():
                sb = (b + NBUF - SD) % NBUF
                pltpu.make_async_copy(rows[sb], acc_sh.at[dbuf[sb]],
                                      ssem[sb]).wait()

            @pl.when(c + PD < NCHUNK)
            def _issue_next_gather():
                nb = (b + PD) % NBUF
                off = pl.multiple_of((c + PD) * CH, 8)
                pltpu.async_copy(xws_hbm.at[srci_v.at[pl.ds(off, CH)]],
                                 rows[nb], gsem[nb])
        return carry

    lax.fori_loop(0, NCHUNK // NBUF, group, 0)
    for k in range(SD):
        b = (NCHUNK - SD + k) % NBUF
        pltpu.make_async_copy(rows[b], acc_sh.at[dbuf[b]], ssem[b]).wait()
    plsc.subcore_barrier()
    pltpu.sync_copy(acc_sh.at[pl.ds(sid * RPT, RPT)],
                    out_hbm.at[pl.ds(cid * NP + sid * RPT, RPT)])


def _prescale_body(nf_ref, wg_ref, wih_ref, whh_ref, bih_ref, bhh_ref,
                   degp_ref, xws_ref):
    W = wg_ref[...]
    gi = lax.dot_general(W, wih_ref[...], (((1,), (1,)), ((), ())),
                         precision=_PREC) + bih_ref[...]
    gh = lax.dot_general(W, whh_ref[...], (((1,), (1,)), ((), ())),
                         precision=_PREC) + bhh_ref[...]
    r = jax.nn.sigmoid(gi[:, :D] + gh[:, :D])
    z = jax.nn.sigmoid(gi[:, D:2 * D] + gh[:, D:2 * D])
    n = jnp.tanh(gi[:, 2 * D:] + r * gh[:, 2 * D:])
    w_ev = (1.0 - z) * n + z * W
    xw = jnp.dot(nf_ref[...], w_ev, precision=_PREC)
    deg = degp_ref[0, :N] + degp_ref[1, :N] + 1.0
    xws_ref[:N] = xw * lax.rsqrt(deg)
    xws_ref[N:] = jnp.zeros((NP - N, D), jnp.float32)


_prescale_call = pl.pallas_call(
    _prescale_body,
    out_shape=jax.ShapeDtypeStruct((NP, D), jnp.float32),
    compiler_params=pltpu.CompilerParams(skip_device_barrier=True),
)


def _out_body(acc_ref, xws_ref, degp_ref, wlin_ref, blin_ref, out_ref):
    deg = degp_ref[0, :N] + degp_ref[1, :N] + 1.0
    h = (acc_ref[0, :N] + acc_ref[1, :N] + xws_ref[:N]) * lax.rsqrt(deg)
    zr = jnp.maximum(h, 0.0)
    out_ref[...] = lax.dot_general(zr, wlin_ref[...], (((1,), (1,)), ((), ())),
                                   precision=_PREC) + blin_ref[...]


_out_call = pl.pallas_call(
    _out_body,
    out_shape=jax.ShapeDtypeStruct((N, D), jnp.float32),
    compiler_params=pltpu.CompilerParams(skip_device_barrier=True),
)


@functools.lru_cache(maxsize=1)
def _sc_kernels():
    mesh = plsc.VectorSubcoreMesh(core_axis_name="c", subcore_axis_name="s")
    deg_kernel = pl.kernel(
        _deg_body,
        out_type=jax.ShapeDtypeStruct((NC * NP,), jnp.float32),
        mesh=mesh,
        compiler_params=pltpu.CompilerParams(needs_layout_passes=False, skip_device_barrier=True),
        scratch_types=[
            pltpu.VMEM((EPW,), jnp.int32),
            pltpu.VMEM((NP,), jnp.float32),
            pltpu.VMEM((RPT,), jnp.float32),
            pltpu.VMEM((RPT,), jnp.float32),
            pltpu.VMEM_SHARED((NS, NP), jnp.float32),
        ],
    )
    agg_kernel = pl.kernel(
        _agg_body,
        out_type=jax.ShapeDtypeStruct((NC * NP, D), jnp.float32),
        mesh=mesh,
        compiler_params=pltpu.CompilerParams(needs_layout_passes=False, skip_device_barrier=True),
        scratch_types=(
            [pltpu.VMEM((EPT,), jnp.int32),
             pltpu.VMEM((EPT,), jnp.int32)]
            + [pltpu.VMEM((CH, D), jnp.float32) for _ in range(NBUF)]
            + [pltpu.VMEM((CH,), jnp.int32) for _ in range(NBUF)]
            + [pltpu.VMEM_SHARED((NP, D), jnp.float32)]
            + [pltpu.SemaphoreType.DMA] * (2 * NBUF)
        ),
    )
    return deg_kernel, agg_kernel


def kernel(node_feat, src, dst, W_gcn, W_ih, W_hh, b_ih, b_hh, W_lin, b_lin):
    src = src.astype(jnp.int32)
    dst = dst.astype(jnp.int32)
    zeros_acc = jnp.zeros((NP, D), jnp.float32)

    srcf = jnp.pad(src, (0, EPAD), constant_values=NP - 1)
    dstf = jnp.pad(dst, (0, EPAD), constant_values=NP - 1)

    _deg_kernel, _agg_kernel = _sc_kernels()
    degp = _deg_kernel(dst).reshape(NC, NP, 1)
    xws = _prescale_call(node_feat.astype(jnp.float32), W_gcn, W_ih, W_hh,
                         b_ih.reshape(1, 3 * D), b_hh.reshape(1, 3 * D), degp)
    acc = _agg_kernel(xws, srcf, dstf, zeros_acc).reshape(NC, NP, D)
    return _out_call(acc, xws, degp, W_lin, b_lin.reshape(1, D))


# speedup vs baseline: 1.3009x; 1.0218x over previous
import functools

import jax
import jax.numpy as jnp
from jax import lax
from jax.experimental import pallas as pl
from jax.experimental.pallas import tpu as pltpu
from jax.experimental.pallas import tpu_sc as plsc

N = 10000
E = 320000
D = 128

NC = 2
NS = 16
NW = NC * NS
EPW = E // NW
CH = 80
NCHUNK = 125
EPT = NCHUNK * CH
EPAD = NW * EPT - E
NP = 10240
RPT = NP // NS

_PREC = jax.lax.Precision.DEFAULT


def _deg_body(dst_hbm, out_hbm, dst_v, deg_v, acc_v, tmp_v, deg_sh):
    cid = lax.axis_index("c")
    sid = lax.axis_index("s")
    wid = sid * NC + cid
    zeros16 = jnp.zeros((16,), jnp.float32)
    ones16 = jnp.ones((16,), jnp.float32)

    def zbody(i, c):
        deg_v[pl.ds(pl.multiple_of(i * 16, 16), 16)] = zeros16
        return c
    lax.fori_loop(0, NP // 16, zbody, 0)

    pltpu.sync_copy(dst_hbm.at[pl.ds(wid * EPW, EPW)], dst_v)

    def ebody(i, c):
        iv = dst_v[pl.ds(pl.multiple_of(i * 16, 16), 16)]
        plsc.addupdate_scatter(deg_v, [iv], ones16)
        return c
    lax.fori_loop(0, EPW // 16, ebody, 0)

    pltpu.sync_copy(deg_v, deg_sh.at[sid])
    plsc.subcore_barrier()

    pltpu.sync_copy(deg_sh.at[0, pl.ds(sid * RPT, RPT)], acc_v)

    def mbody(j, c):
        pltpu.sync_copy(deg_sh.at[j, pl.ds(sid * RPT, RPT)], tmp_v)

        def abody(i, c2):
            sl = pl.ds(pl.multiple_of(i * 16, 16), 16)
            acc_v[sl] = acc_v[sl] + tmp_v[sl]
            return c2
        lax.fori_loop(0, RPT // 16, abody, 0)
        return c
    lax.fori_loop(1, NS, mbody, 0)
    pltpu.sync_copy(acc_v, out_hbm.at[pl.ds(cid * NP + sid * RPT, RPT)])


def _agg_body(xws_hbm, srcf_hbm, dstf_hbm, zeros_hbm, out_hbm,
              src_v, dst_v, rows_v, acc_sh, sem):
    cid = lax.axis_index("c")
    sid = lax.axis_index("s")
    wid = sid * NC + cid
    pltpu.sync_copy(zeros_hbm.at[pl.ds(sid * RPT, RPT)],
                    acc_sh.at[pl.ds(sid * RPT, RPT)])
    plsc.subcore_barrier()
    base = wid * EPT

    def body(c, carry):
        off = pl.multiple_of(base + c * CH, 8)
        pltpu.sync_copy(srcf_hbm.at[pl.ds(off, CH)], src_v)
        pltpu.sync_copy(dstf_hbm.at[pl.ds(off, CH)], dst_v)
        pltpu.async_copy(xws_hbm.at[src_v], rows_v, sem).wait()
        pltpu.sync_copy(rows_v, acc_sh.at[dst_v], add=True)
        return carry

    lax.fori_loop(0, NCHUNK, body, 0)
    plsc.subcore_barrier()
    pltpu.sync_copy(acc_sh.at[pl.ds(sid * RPT, RPT)],
                    out_hbm.at[pl.ds(cid * NP + sid * RPT, RPT)])


def _prescale_body(nf_ref, wg_ref, wih_ref, whh_ref, bih_ref, bhh_ref,
                   degp_ref, xws_ref):
    W = wg_ref[...]
    gi = lax.dot_general(W, wih_ref[...], (((1,), (1,)), ((), ())),
                         precision=_PREC) + bih_ref[...]
    gh = lax.dot_general(W, whh_ref[...], (((1,), (1,)), ((), ())),
                         precision=_PREC) + bhh_ref[...]
    r = jax.nn.sigmoid(gi[:, :D] + gh[:, :D])
    z = jax.nn.sigmoid(gi[:, D:2 * D] + gh[:, D:2 * D])
    n = jnp.tanh(gi[:, 2 * D:] + r * gh[:, 2 * D:])
    w_ev = (1.0 - z) * n + z * W
    xw = jnp.dot(nf_ref[...], w_ev, precision=_PREC)
    deg = degp_ref[0, :N] + degp_ref[1, :N] + 1.0
    xws_ref[:N] = xw * lax.rsqrt(deg)
    xws_ref[N:] = jnp.zeros((NP - N, D), jnp.float32)


_prescale_call = pl.pallas_call(
    _prescale_body,
    out_shape=jax.ShapeDtypeStruct((NP, D), jnp.float32),
    compiler_params=pltpu.CompilerParams(skip_device_barrier=True),
)


def _out_body(acc_ref, xws_ref, degp_ref, wlin_ref, blin_ref, out_ref):
    deg = degp_ref[0, :N] + degp_ref[1, :N] + 1.0
    h = (acc_ref[0, :N] + acc_ref[1, :N] + xws_ref[:N]) * lax.rsqrt(deg)
    zr = jnp.maximum(h, 0.0)
    out_ref[...] = lax.dot_general(zr, wlin_ref[...], (((1,), (1,)), ((), ())),
                                   precision=_PREC) + blin_ref[...]


_out_call = pl.pallas_call(
    _out_body,
    out_shape=jax.ShapeDtypeStruct((N, D), jnp.float32),
    compiler_params=pltpu.CompilerParams(skip_device_barrier=True),
)


@functools.lru_cache(maxsize=1)
def _sc_kernels():
    mesh = plsc.VectorSubcoreMesh(core_axis_name="c", subcore_axis_name="s")
    deg_kernel = pl.kernel(
        _deg_body,
        out_type=jax.ShapeDtypeStruct((NC * NP,), jnp.float32),
        mesh=mesh,
        compiler_params=pltpu.CompilerParams(needs_layout_passes=False, skip_device_barrier=True),
        scratch_types=[
            pltpu.VMEM((EPW,), jnp.int32),
            pltpu.VMEM((NP,), jnp.float32),
            pltpu.VMEM((RPT,), jnp.float32),
            pltpu.VMEM((RPT,), jnp.float32),
            pltpu.VMEM_SHARED((NS, NP), jnp.float32),
        ],
    )
    agg_kernel = pl.kernel(
        _agg_body,
        out_type=jax.ShapeDtypeStruct((NC * NP, D), jnp.float32),
        mesh=mesh,
        compiler_params=pltpu.CompilerParams(skip_device_barrier=True),
        scratch_types=[
            pltpu.VMEM((CH,), jnp.int32),
            pltpu.VMEM((CH,), jnp.int32),
            pltpu.VMEM((CH, D), jnp.float32),
            pltpu.VMEM_SHARED((NP, D), jnp.float32),
            pltpu.SemaphoreType.DMA,
        ],
    )
    return deg_kernel, agg_kernel


def kernel(node_feat, src, dst, W_gcn, W_ih, W_hh, b_ih, b_hh, W_lin, b_lin):
    src = src.astype(jnp.int32)
    dst = dst.astype(jnp.int32)
    zeros_acc = jnp.zeros((NP, D), jnp.float32)

    srcf = jnp.pad(src, (0, EPAD), constant_values=NP - 1)
    dstf = jnp.pad(dst, (0, EPAD), constant_values=NP - 1)

    _deg_kernel, _agg_kernel = _sc_kernels()
    degp = _deg_kernel(dst).reshape(NC, NP, 1)
    xws = _prescale_call(node_feat.astype(jnp.float32), W_gcn, W_ih, W_hh,
                         b_ih.reshape(1, 3 * D), b_hh.reshape(1, 3 * D), degp)
    acc = _agg_kernel(xws, srcf, dstf, zeros_acc).reshape(NC, NP, D)
    return _out_call(acc, xws, degp, W_lin, b_lin.reshape(1, D))
